# baseline (speedup 1.0000x reference)
import functools

import jax
import jax.numpy as jnp
from jax import lax
from jax.experimental import pallas as pl
from jax.experimental.pallas import tpu as pltpu
from jax.experimental.pallas import tpu_sc as plsc

_B = 16384
_S = 26
_DIM = 64
_V = 1000000
_TOTAL = _B * _S
_NW = 32
_PER_W = _TOTAL // _NW
_CHUNK = 128
_NCHUNK = _PER_W // _CHUNK
_NBUF = 8
_NCOL = _V // 128
_COL_W = _NCOL // _NW
_NEXTRA = _NCOL - _NW * _COL_W
_TAIL = _V - _NCOL * 128
_VP = (_V + _TAIL) // 2


def _build_format():
    info = plsc.get_sparse_core_info()
    nc = info.num_cores
    mesh = plsc.VectorSubcoreMesh(core_axis_name="c", subcore_axis_name="s")

    @functools.partial(
        pl.kernel,
        mesh=mesh,
        out_type=jax.ShapeDtypeStruct((_VP, 128), jnp.float32),
        scratch_types=[
            pltpu.VMEM((6, _DIM, 128), jnp.float32),
            pltpu.VMEM((6, _DIM, 128), jnp.float32),
            pltpu.VMEM((_TAIL // 2, 128), jnp.float32),
            pltpu.SemaphoreType.DMA((6,)),
            pltpu.SemaphoreType.DMA((6,)),
        ],
        compiler_params=pltpu.CompilerParams(
            use_tc_tiling_on_sc=True, needs_layout_passes=False
        ),
    )
    def format_kernel(tab_hbm, tail_hbm, out_hbm, fbuf, rbuf, tailbuf,
                      isem, osem):
        wid = lax.axis_index("s") * nc + lax.axis_index("c")

        iota16 = lax.iota(jnp.int32, 16)
        lanes = [iota16 + 16 * g for g in range(8)]
        pair_rows = [(iota16 + 16 * g) >> 1 for g in range(8)]
        odd_offs = [((iota16 & 1) << 6) for _ in range(8)]

        def col_sel(t):
            return jnp.where(t < _COL_W, wid * _COL_W + t,
                             _NW * _COL_W + wid)

        def istart(b, col):
            off = pl.multiple_of(col * 128, 128)
            pltpu.async_copy(
                tab_hbm.at[:, pl.ds(off, 128)], fbuf.at[b], isem.at[b]
            )

        def iwait(b):
            pltpu.make_async_copy(
                tab_hbm.at[:, pl.ds(0, 128)], fbuf.at[b], isem.at[b]
            ).wait()

        def ostart(b, col):
            off = pl.multiple_of(col * 64, 64)
            pltpu.async_copy(
                rbuf.at[b], out_hbm.at[pl.ds(off, 64)], osem.at[b]
            )

        def owait(b):
            pltpu.make_async_copy(
                rbuf.at[b], out_hbm.at[pl.ds(0, 64)], osem.at[b]
            ).wait()

        def transpose(b):
            @pl.loop(0, 16, unroll=4)
            def _(o, b=b):
                rot = (iota16 + o) & 15
                for dbase in range(0, _DIM, 16):
                    dp = rot + dbase
                    vals = [
                        plsc.load_gather(fbuf.at[b], [dp, lanes[g]])
                        for g in range(8)
                    ]
                    for g in range(8):
                        plsc.store_scatter(
                            rbuf.at[b], [pair_rows[g], odd_offs[g] + dp], vals[g]
                        )

        @pl.when(wid == _NW - 1)
        def _():
            pltpu.async_copy(tail_hbm, tailbuf, isem.at[0])
            pltpu.make_async_copy(tail_hbm, tailbuf, isem.at[0]).wait()
            pltpu.async_copy(
                tailbuf, out_hbm.at[pl.ds(_NCOL * 64, _TAIL // 2)], osem.at[0]
            )
            pltpu.make_async_copy(
                tailbuf, out_hbm.at[pl.ds(_NCOL * 64, _TAIL // 2)], osem.at[0]
            ).wait()

        ncols = jnp.where(wid < _NEXTRA, _COL_W + 1, _COL_W)

        for b in range(6):
            istart(b, col_sel(b))

        @pl.loop(0, _COL_W + 6, step=6)
        def _(t0):
            for b in range(6):
                t = t0 + b

                @pl.when(t < ncols)
                def _(t=t, b=b):
                    iwait(b)

                    @pl.when(t >= 6)
                    def _():
                        owait(b)

                    transpose(b)

                    @pl.when(t + 6 < ncols)
                    def _():
                        istart(b, col_sel(t + 6))

                    ostart(b, col_sel(t))

        for b in range(6):
            owait(b)

    return format_kernel


def _build_gather():
    info = plsc.get_sparse_core_info()
    nc = info.num_cores
    mesh = plsc.VectorSubcoreMesh(core_axis_name="c", subcore_axis_name="s")

    @functools.partial(
        pl.kernel,
        mesh=mesh,
        out_type=jax.ShapeDtypeStruct((_S, 8, _B // _CHUNK, 8, _CHUNK), jnp.float32),
        scratch_types=[
            pltpu.VMEM((_PER_W,), jnp.int32),
            pltpu.VMEM((_NBUF, _CHUNK, _DIM), jnp.float32),
            pltpu.VMEM((2, 8, 8, _CHUNK), jnp.float32),
            pltpu.SemaphoreType.DMA((_NBUF,)),
            pltpu.SemaphoreType.DMA((2,)),
        ],
        compiler_params=pltpu.CompilerParams(
            use_tc_tiling_on_sc=False, needs_layout_passes=False
        ),
    )
    def gather_kernel(idx_hbm, table_hbm, out_hbm, rowv, buf, tbuf,
                      gsem, osem):
        wid = lax.axis_index("s") * nc + lax.axis_index("c")
        base = wid * _PER_W
        pltpu.sync_copy(idx_hbm.at[pl.ds(base, _PER_W)], rowv)

        lanes = [lax.iota(jnp.int32, 16) + 16 * g for g in range(8)]

        def gstart(b, c):
            pltpu.async_copy(
                table_hbm.at[rowv.at[pl.ds(c * _CHUNK, _CHUNK)]],
                buf.at[b],
                gsem.at[b],
            )

        def gwait(b):
            pltpu.make_async_copy(
                table_hbm.at[rowv.at[pl.ds(0, _CHUNK)]], buf.at[b], gsem.at[b]
            ).wait()

        def out_slice(c):
            gchunk = wid * _NCHUNK + c
            j = gchunk >> 7
            tc = gchunk & 127
            return out_hbm.at[j, :, tc]

        def ostart(tb, c):
            pltpu.async_copy(tbuf.at[tb], out_slice(c), osem.at[tb])

        def owait(tb):
            pltpu.make_async_copy(tbuf.at[tb], out_slice(0), osem.at[tb]).wait()

        iota16 = lax.iota(jnp.int32, 16)

        def transpose(b, tb, c):
            del c

            @pl.loop(0, 16, unroll=4)
            def _(o, b=b, tb=tb):
                rot = (iota16 + o) & 15
                for dbase in range(0, _DIM, 16):
                    dp = rot + dbase
                    trv = dp >> 3
                    sv = dp & 7
                    vals = [
                        plsc.load_gather(buf.at[b], [lanes[g], dp])
                        for g in range(8)
                    ]
                    for g in range(8):
                        plsc.store_scatter(
                            tbuf.at[tb], [trv, sv, lanes[g]], vals[g]
                        )

        for b in range(_NBUF):
            gstart(b, b)
        for b in range(_NBUF):
            gwait(b)
            if b >= 2:
                owait(b & 1)
            transpose(b, b & 1, b)
            gstart(b, b + _NBUF)
            ostart(b & 1, b)

        @pl.loop(_NBUF, _NCHUNK - _NBUF, step=_NBUF)
        def _(c):
            for b in range(_NBUF):
                gwait(b)
                owait(b & 1)
                transpose(b, b & 1, c + b)
                gstart(b, c + b + _NBUF)
                ostart(b & 1, c + b)

        for b in range(_NBUF):
            gwait(b)
            owait(b & 1)
            transpose(b, b & 1, _NCHUNK - _NBUF + b)
            ostart(b & 1, _NCHUNK - _NBUF + b)
        for tb in range(2):
            owait(tb)

    return gather_kernel


_format = _build_format()
_gather = _build_gather()


def kernel(indices, codewords):
    idx_flat = indices.T.reshape(-1).astype(jnp.int32)
    tab_t = jnp.transpose(codewords)
    tail = lax.slice(codewords, (_NCOL * 128, 0), (_V, _DIM))
    tail2 = tail.reshape(_TAIL // 2, 128)
    table2 = _format(tab_t, tail2)
    table3 = table2.reshape(_VP * 2, _DIM)
    out5 = _gather(idx_flat, table3)
    return jnp.transpose(out5, (2, 4, 0, 1, 3)).reshape(_B, _S, _DIM)


# speedup vs baseline: 1.0895x; 1.0895x over previous
import functools

import jax
import jax.numpy as jnp
from jax import lax
from jax.experimental import pallas as pl
from jax.experimental.pallas import tpu as pltpu
from jax.experimental.pallas import tpu_sc as plsc

_B = 16384
_S = 26
_DIM = 64
_V = 1000000
_TOTAL = _B * _S
_NW = 32
_PER_W = _TOTAL // _NW
_CHUNK = 128
_NCHUNK = _PER_W // _CHUNK
_NBUF = 8
_NCOL = _V // 128
_COL_W = _NCOL // _NW
_NEXTRA = _NCOL - _NW * _COL_W
_TAIL = _V - _NCOL * 128
_VP = (_V + _TAIL) // 2


def _build_format():
    info = plsc.get_sparse_core_info()
    nc = info.num_cores
    mesh = plsc.VectorSubcoreMesh(core_axis_name="c", subcore_axis_name="s")

    @functools.partial(
        pl.kernel,
        mesh=mesh,
        out_type=jax.ShapeDtypeStruct((_VP, 128), jnp.float32),
        scratch_types=[
            pltpu.VMEM((6, _DIM, 128), jnp.float32),
            pltpu.VMEM((6, _DIM, 128), jnp.float32),
            pltpu.VMEM((_TAIL // 2, 128), jnp.float32),
            pltpu.SemaphoreType.DMA((6,)),
            pltpu.SemaphoreType.DMA((6,)),
        ],
        compiler_params=pltpu.CompilerParams(
            use_tc_tiling_on_sc=True, needs_layout_passes=False
        ),
    )
    def format_kernel(tab_hbm, tail_hbm, out_hbm, fbuf, rbuf, tailbuf,
                      isem, osem):
        wid = lax.axis_index("s") * nc + lax.axis_index("c")

        iota16 = lax.iota(jnp.int32, 16)
        lanes = [iota16 + 16 * g for g in range(8)]
        pair_rows = [(iota16 + 16 * g) >> 1 for g in range(8)]
        odd_offs = [((iota16 & 1) << 6) for _ in range(8)]

        def col_sel(t):
            return jnp.where(t < _COL_W, wid * _COL_W + t,
                             _NW * _COL_W + wid)

        def istart(b, col):
            off = pl.multiple_of(col * 128, 128)
            pltpu.async_copy(
                tab_hbm.at[:, pl.ds(off, 128)], fbuf.at[b], isem.at[b]
            )

        def iwait(b):
            pltpu.make_async_copy(
                tab_hbm.at[:, pl.ds(0, 128)], fbuf.at[b], isem.at[b]
            ).wait()

        def ostart(b, col):
            off = pl.multiple_of(col * 64, 64)
            pltpu.async_copy(
                rbuf.at[b], out_hbm.at[pl.ds(off, 64)], osem.at[b]
            )

        def owait(b):
            pltpu.make_async_copy(
                rbuf.at[b], out_hbm.at[pl.ds(0, 64)], osem.at[b]
            ).wait()

        def transpose(b):
            @pl.loop(0, 16, unroll=2)
            def _(o, b=b):
                rot = (iota16 + o) & 15
                for dbase in range(0, _DIM, 16):
                    dp = rot + dbase
                    vals = [
                        plsc.load_gather(fbuf.at[b], [dp, lanes[g]])
                        for g in range(8)
                    ]
                    for g in range(8):
                        plsc.store_scatter(
                            rbuf.at[b], [pair_rows[g], odd_offs[g] + dp], vals[g]
                        )

        @pl.when(wid == _NW - 1)
        def _():
            pltpu.async_copy(tail_hbm, tailbuf, isem.at[0])
            pltpu.make_async_copy(tail_hbm, tailbuf, isem.at[0]).wait()
            pltpu.async_copy(
                tailbuf, out_hbm.at[pl.ds(_NCOL * 64, _TAIL // 2)], osem.at[0]
            )
            pltpu.make_async_copy(
                tailbuf, out_hbm.at[pl.ds(_NCOL * 64, _TAIL // 2)], osem.at[0]
            ).wait()

        ncols = jnp.where(wid < _NEXTRA, _COL_W + 1, _COL_W)

        for b in range(6):
            istart(b, col_sel(b))

        @pl.loop(0, _COL_W + 6, step=6)
        def _(t0):
            for b in range(6):
                t = t0 + b

                @pl.when(t < ncols)
                def _(t=t, b=b):
                    iwait(b)

                    @pl.when(t >= 6)
                    def _():
                        owait(b)

                    transpose(b)

                    @pl.when(t + 6 < ncols)
                    def _():
                        istart(b, col_sel(t + 6))

                    ostart(b, col_sel(t))

        for b in range(6):
            owait(b)

    return format_kernel


def _build_gather():
    info = plsc.get_sparse_core_info()
    nc = info.num_cores
    mesh = plsc.VectorSubcoreMesh(core_axis_name="c", subcore_axis_name="s")

    @functools.partial(
        pl.kernel,
        mesh=mesh,
        out_type=jax.ShapeDtypeStruct((_S, 8, _B // _CHUNK, 8, _CHUNK), jnp.float32),
        scratch_types=[
            pltpu.VMEM((_PER_W,), jnp.int32),
            pltpu.VMEM((_NBUF, _CHUNK, _DIM), jnp.float32),
            pltpu.VMEM((2, 8, 8, _CHUNK), jnp.float32),
            pltpu.SemaphoreType.DMA((_NBUF,)),
            pltpu.SemaphoreType.DMA((2,)),
        ],
        compiler_params=pltpu.CompilerParams(
            use_tc_tiling_on_sc=False, needs_layout_passes=False
        ),
    )
    def gather_kernel(idx_hbm, table_hbm, out_hbm, rowv, buf, tbuf,
                      gsem, osem):
        wid = lax.axis_index("s") * nc + lax.axis_index("c")
        base = wid * _PER_W
        pltpu.sync_copy(idx_hbm.at[pl.ds(base, _PER_W)], rowv)

        lanes = [lax.iota(jnp.int32, 16) + 16 * g for g in range(8)]

        def gstart(b, c):
            pltpu.async_copy(
                table_hbm.at[rowv.at[pl.ds(c * _CHUNK, _CHUNK)]],
                buf.at[b],
                gsem.at[b],
            )

        def gwait(b):
            pltpu.make_async_copy(
                table_hbm.at[rowv.at[pl.ds(0, _CHUNK)]], buf.at[b], gsem.at[b]
            ).wait()

        def out_slice(c):
            gchunk = wid * _NCHUNK + c
            j = gchunk >> 7
            tc = gchunk & 127
            return out_hbm.at[j, :, tc]

        def ostart(tb, c):
            pltpu.async_copy(tbuf.at[tb], out_slice(c), osem.at[tb])

        def owait(tb):
            pltpu.make_async_copy(tbuf.at[tb], out_slice(0), osem.at[tb]).wait()

        iota16 = lax.iota(jnp.int32, 16)

        def transpose(b, tb, c):
            del c

            @pl.loop(0, 16, unroll=2)
            def _(o, b=b, tb=tb):
                rot = (iota16 + o) & 15
                for dbase in range(0, _DIM, 16):
                    dp = rot + dbase
                    trv = dp >> 3
                    sv = dp & 7
                    vals = [
                        plsc.load_gather(buf.at[b], [lanes[g], dp])
                        for g in range(8)
                    ]
                    for g in range(8):
                        plsc.store_scatter(
                            tbuf.at[tb], [trv, sv, lanes[g]], vals[g]
                        )

        for b in range(_NBUF):
            gstart(b, b)
        for b in range(_NBUF):
            gwait(b)
            if b >= 2:
                owait(b & 1)
            transpose(b, b & 1, b)
            gstart(b, b + _NBUF)
            ostart(b & 1, b)

        @pl.loop(_NBUF, _NCHUNK - _NBUF, step=_NBUF)
        def _(c):
            for b in range(_NBUF):
                gwait(b)
                owait(b & 1)
                transpose(b, b & 1, c + b)
                gstart(b, c + b + _NBUF)
                ostart(b & 1, c + b)

        for b in range(_NBUF):
            gwait(b)
            owait(b & 1)
            transpose(b, b & 1, _NCHUNK - _NBUF + b)
            ostart(b & 1, _NCHUNK - _NBUF + b)
        for tb in range(2):
            owait(tb)

    return gather_kernel


_format = _build_format()
_gather = _build_gather()


def kernel(indices, codewords):
    idx_flat = indices.T.reshape(-1).astype(jnp.int32)
    tab_t = jnp.transpose(codewords)
    tail = lax.slice(codewords, (_NCOL * 128, 0), (_V, _DIM))
    tail2 = tail.reshape(_TAIL // 2, 128)
    table2 = _format(tab_t, tail2)
    table3 = table2.reshape(_VP * 2, _DIM)
    out5 = _gather(idx_flat, table3)
    return jnp.transpose(out5, (2, 4, 0, 1, 3)).reshape(_B, _S, _DIM)
